# baseline (speedup 1.0000x reference)
import functools

import jax
import jax.numpy as jnp
from jax import lax
from jax.experimental import pallas as pl
from jax.experimental.pallas import tpu as pltpu
from jax.experimental.pallas import tpu_sc as plsc

N = 10000
E = 320000
D_IN = 128
D_HID = 16
D_OUT = 128

NC = 2
NS = 16
NW = NC * NS
EW = E // NW
C = 1000
N_PAD = 10240
RPT = N_PAD // NS


def _matmul_body(x_ref, w_ref, o_ref):
    o_ref[...] = jnp.dot(x_ref[...], w_ref[...], preferred_element_type=jnp.float32)


def _project(features, W1):
    return pl.pallas_call(
        _matmul_body,
        out_shape=jax.ShapeDtypeStruct((N, D_HID), jnp.float32),
    )(features, W1)


def _edge_body(f1, edges, zrows, zdeg, h1_out, deg_out,
               s_idx, d_idx, rows, hist,
               sem0, sem1, semh0, semh1, h1_sh):
    sems = [sem0, sem1]
    sem_sh = [semh0, semh1]
    cid = lax.axis_index("c")
    sid = lax.axis_index("s")
    wid = sid * NC + cid

    pltpu.sync_copy(zrows, rows.at[0, pl.ds(0, RPT)])
    pltpu.sync_copy(zdeg.at[pl.ds(0, N_PAD)], hist)
    pltpu.sync_copy(rows.at[0, pl.ds(0, RPT)], h1_sh.at[pl.ds(sid * RPT, RPT)])
    plsc.subcore_barrier()

    base = wid * EW
    n_chunks = EW // C
    ones16 = jnp.ones((16,), jnp.float32)

    gathers = [None, None]
    scat_h = [None, None]

    def load(i):
        b = i % 2
        off = base + i * C
        pltpu.sync_copy(edges.at[0, pl.ds(off, C)], s_idx.at[b])
        pltpu.sync_copy(edges.at[1, pl.ds(off, C)], d_idx.at[b])
        gathers[b] = pltpu.async_copy(f1.at[s_idx.at[b]], rows.at[b], sems[b])

    load(0)
    for i in range(n_chunks):
        b = i % 2
        if i >= 1:
            scat_h[1 - b].wait()
        if i + 1 < n_chunks:
            load(i + 1)
        gathers[b].wait()
        scat_h[b] = pltpu.async_copy(rows.at[b], h1_sh.at[d_idx.at[b]],
                                     sem_sh[b], add=True)

        def hist_body(k, carry):
            off = pl.multiple_of(k * 16, 16)
            idx = s_idx[b, pl.ds(off, 16)]
            plsc.addupdate_scatter(hist, [idx], ones16)
            return carry

        lax.fori_loop(0, C // 16, hist_body, 0, unroll=4)
    lb = (n_chunks - 1) % 2
    scat_h[lb].wait()
    plsc.subcore_barrier()

    pltpu.sync_copy(hist, deg_out.at[cid, sid])
    pltpu.sync_copy(h1_sh.at[pl.ds(sid * RPT, RPT)],
                    h1_out.at[cid, pl.ds(sid * RPT, RPT)])


def _edge_pass(f1, edges):
    mesh = plsc.VectorSubcoreMesh(
        core_axis_name="c", subcore_axis_name="s",
        num_cores=NC, num_subcores=NS)
    run = functools.partial(
        pl.kernel,
        out_type=(
            jax.ShapeDtypeStruct((NC, N_PAD, D_HID), jnp.float32),
            jax.ShapeDtypeStruct((NC, NS, N_PAD), jnp.float32),
        ),
        mesh=mesh,
        scratch_types=[
            pltpu.VMEM((2, C), jnp.int32),
            pltpu.VMEM((2, C), jnp.int32),
            pltpu.VMEM((2, C, D_HID), jnp.float32),
            pltpu.VMEM((N_PAD,), jnp.float32),
            pltpu.SemaphoreType.DMA,
            pltpu.SemaphoreType.DMA,
            pltpu.SemaphoreType.DMA,
            pltpu.SemaphoreType.DMA,
            pltpu.MemorySpace.VMEM_SHARED((N_PAD, D_HID), jnp.float32),
        ],
        compiler_params=pltpu.CompilerParams(
            use_tc_tiling_on_sc=False, needs_layout_passes=False),
    )(_edge_body)
    zrows = jnp.zeros((RPT, D_HID), jnp.float32)
    zdeg = jnp.zeros((N_PAD + 16,), jnp.float32)
    return run(f1, edges, zrows, zdeg)


RPR = N_PAD // (NC * NS)


def _reduce_body(h1p, degp, b1, s_out,
                 a_v, b_v, d_v, b1_v, acc_v, sem, s_sh):
    cid = lax.axis_index("c")
    sid = lax.axis_index("s")
    base = cid * (N_PAD // NC) + sid * RPR

    cps = [
        pltpu.async_copy(b1, b1_v, sem),
        pltpu.async_copy(h1p.at[0, pl.ds(base, RPR)], a_v, sem),
        pltpu.async_copy(h1p.at[1, pl.ds(base, RPR)], b_v, sem),
    ]
    cps += [
        pltpu.async_copy(degp.at[w // NS, w % NS, pl.ds(base, RPR)],
                         d_v.at[w], sem)
        for w in range(NW)
    ]
    for cp in cps:
        cp.wait()

    b1v = b1_v[...]

    def body(g, acc):
        off = pl.multiple_of(g * 16, 16)
        dv = d_v[0, pl.ds(off, 16)]
        for w in range(1, NW):
            dv = dv + d_v[w, pl.ds(off, 16)]
        for j in range(16):
            r = g * 16 + j
            x = jnp.maximum(a_v[r, :] + b_v[r, :] + b1v, 0.0)
            acc = acc + x * dv[j]
        return acc
    acc = lax.fori_loop(0, RPR // 16, body, jnp.zeros((16,), jnp.float32))
    acc_v[...] = acc

    pltpu.sync_copy(acc_v, s_sh.at[sid])
    plsc.subcore_barrier()

    @pl.when(sid == 0)
    def _():
        def comb(t, tot):
            pltpu.sync_copy(s_sh.at[t], acc_v)
            return tot + acc_v[...]
        tot = lax.fori_loop(0, NS, comb, jnp.zeros((16,), jnp.float32))
        acc_v[...] = tot
        pltpu.sync_copy(acc_v, s_out.at[cid])


def _reduce(h1p, degp, b1):
    mesh = plsc.VectorSubcoreMesh(
        core_axis_name="c", subcore_axis_name="s",
        num_cores=NC, num_subcores=NS)
    run = functools.partial(
        pl.kernel,
        out_type=jax.ShapeDtypeStruct((NC, 16), jnp.float32),
        mesh=mesh,
        scratch_types=[
            pltpu.VMEM((RPR, D_HID), jnp.float32),
            pltpu.VMEM((RPR, D_HID), jnp.float32),
            pltpu.VMEM((NW, RPR), jnp.float32),
            pltpu.VMEM((16,), jnp.float32),
            pltpu.VMEM((16,), jnp.float32),
            pltpu.SemaphoreType.DMA,
            pltpu.MemorySpace.VMEM_SHARED((NS, 16), jnp.float32),
        ],
        compiler_params=pltpu.CompilerParams(use_tc_tiling_on_sc=False),
    )(_reduce_body)
    return run(h1p, degp, b1)


def _head_body(sp_ref, w2_ref, b2_ref, w3_ref, b3_ref, o_ref):
    s = sp_ref[0] + sp_ref[1]
    t = jnp.dot(s[None, :], w2_ref[...], preferred_element_type=jnp.float32)
    t = t + jnp.float32(N) * b2_ref[...]
    o = jnp.dot(t, w3_ref[...], preferred_element_type=jnp.float32) + b3_ref[...]
    o_ref[...] = jnp.maximum(o, 0.0)


def _head(sp, W2, b2, W3, b3):
    return pl.pallas_call(
        _head_body,
        out_shape=jax.ShapeDtypeStruct((1, D_OUT), jnp.float32),
    )(sp, W2, b2, W3, b3)


def kernel(features, edge_index, W1, b1, W2, b2, W3, b3):
    f1 = _project(features, W1)
    h1p, degp = _edge_pass(f1, edge_index)
    sp = _reduce(h1p, degp, b1)
    return _head(sp, W2, b2, W3, b3)


# speedup vs baseline: 1.0589x; 1.0589x over previous
import functools

import jax
import jax.numpy as jnp
from jax import lax
from jax.experimental import pallas as pl
from jax.experimental.pallas import tpu as pltpu
from jax.experimental.pallas import tpu_sc as plsc

N = 10000
E = 320000
D_IN = 128
D_HID = 16
D_OUT = 128

NC = 2
NS = 16
NW = NC * NS
EW = E // NW
C = 2000
NBUF = 3
N_PAD = 10240
RPT = N_PAD // NS


def _matmul_body(x_ref, w_ref, o_ref):
    o_ref[...] = jnp.dot(x_ref[...], w_ref[...], preferred_element_type=jnp.float32)


def _project(features, W1):
    return pl.pallas_call(
        _matmul_body,
        out_shape=jax.ShapeDtypeStruct((N, D_HID), jnp.float32),
    )(features, W1)


def _edge_body(f1, edges, zrows, zdeg, h1_out, deg_out,
               s_idx, d_idx, rows, hist,
               sem0, sem1, sem2, semh0, semh1, semh2, h1_sh):
    sems = [sem0, sem1, sem2]
    sem_sh = [semh0, semh1, semh2]
    cid = lax.axis_index("c")
    sid = lax.axis_index("s")
    wid = sid * NC + cid

    pltpu.sync_copy(zrows, rows.at[0, pl.ds(0, RPT)])
    pltpu.sync_copy(zdeg.at[pl.ds(0, N_PAD)], hist)
    pltpu.sync_copy(rows.at[0, pl.ds(0, RPT)], h1_sh.at[pl.ds(sid * RPT, RPT)])
    plsc.subcore_barrier()

    base = wid * EW
    n_chunks = EW // C
    ones16 = jnp.ones((16,), jnp.float32)

    gathers = [None] * NBUF
    scat_h = [None] * NBUF

    def load(i):
        b = i % NBUF
        off = base + i * C
        pltpu.sync_copy(edges.at[0, pl.ds(off, C)], s_idx.at[b])
        pltpu.sync_copy(edges.at[1, pl.ds(off, C)], d_idx.at[b])
        gathers[b] = pltpu.async_copy(f1.at[s_idx.at[b]], rows.at[b], sems[b])

    for i in range(min(2, n_chunks)):
        load(i)
    for i in range(n_chunks):
        b = i % NBUF
        if i + 2 < n_chunks:
            if i >= 1:
                scat_h[(i - 1) % NBUF].wait()
            load(i + 2)
        gathers[b].wait()
        scat_h[b] = pltpu.async_copy(rows.at[b], h1_sh.at[d_idx.at[b]],
                                     sem_sh[b], add=True)

        def hist_body(k, carry):
            off = pl.multiple_of(k * 16, 16)
            idx = s_idx[b, pl.ds(off, 16)]
            plsc.addupdate_scatter(hist, [idx], ones16)
            return carry

        lax.fori_loop(0, C // 16, hist_body, 0, unroll=4)
    for k in range(max(0, n_chunks - 3), n_chunks):
        scat_h[k % NBUF].wait()
    plsc.subcore_barrier()

    pltpu.sync_copy(hist, deg_out.at[cid, sid])
    pltpu.sync_copy(h1_sh.at[pl.ds(sid * RPT, RPT)],
                    h1_out.at[cid, pl.ds(sid * RPT, RPT)])


def _edge_pass(f1, edges):
    mesh = plsc.VectorSubcoreMesh(
        core_axis_name="c", subcore_axis_name="s",
        num_cores=NC, num_subcores=NS)
    run = functools.partial(
        pl.kernel,
        out_type=(
            jax.ShapeDtypeStruct((NC, N_PAD, D_HID), jnp.float32),
            jax.ShapeDtypeStruct((NC, NS, N_PAD), jnp.float32),
        ),
        mesh=mesh,
        scratch_types=[
            pltpu.VMEM((NBUF, C), jnp.int32),
            pltpu.VMEM((NBUF, C), jnp.int32),
            pltpu.VMEM((NBUF, C, D_HID), jnp.float32),
            pltpu.VMEM((N_PAD,), jnp.float32),
            pltpu.SemaphoreType.DMA,
            pltpu.SemaphoreType.DMA,
            pltpu.SemaphoreType.DMA,
            pltpu.SemaphoreType.DMA,
            pltpu.SemaphoreType.DMA,
            pltpu.SemaphoreType.DMA,
            pltpu.MemorySpace.VMEM_SHARED((N_PAD, D_HID), jnp.float32),
        ],
        compiler_params=pltpu.CompilerParams(
            use_tc_tiling_on_sc=False, needs_layout_passes=False),
    )(_edge_body)
    zrows = jnp.zeros((RPT, D_HID), jnp.float32)
    zdeg = jnp.zeros((N_PAD + 16,), jnp.float32)
    return run(f1, edges, zrows, zdeg)


RPR = N_PAD // (NC * NS)


def _reduce_body(h1p, degp, b1, s_out,
                 a_v, b_v, d_v, b1_v, acc_v, sem, s_sh):
    cid = lax.axis_index("c")
    sid = lax.axis_index("s")
    base = cid * (N_PAD // NC) + sid * RPR

    cps = [
        pltpu.async_copy(b1, b1_v, sem),
        pltpu.async_copy(h1p.at[0, pl.ds(base, RPR)], a_v, sem),
        pltpu.async_copy(h1p.at[1, pl.ds(base, RPR)], b_v, sem),
    ]
    cps += [
        pltpu.async_copy(degp.at[w // NS, w % NS, pl.ds(base, RPR)],
                         d_v.at[w], sem)
        for w in range(NW)
    ]
    for cp in cps:
        cp.wait()

    b1v = b1_v[...]

    def body(g, acc):
        off = pl.multiple_of(g * 16, 16)
        dv = d_v[0, pl.ds(off, 16)]
        for w in range(1, NW):
            dv = dv + d_v[w, pl.ds(off, 16)]
        for j in range(16):
            r = g * 16 + j
            x = jnp.maximum(a_v[r, :] + b_v[r, :] + b1v, 0.0)
            acc = acc + x * dv[j]
        return acc
    acc = lax.fori_loop(0, RPR // 16, body, jnp.zeros((16,), jnp.float32))
    acc_v[...] = acc

    pltpu.sync_copy(acc_v, s_sh.at[sid])
    plsc.subcore_barrier()

    @pl.when(sid == 0)
    def _():
        def comb(t, tot):
            pltpu.sync_copy(s_sh.at[t], acc_v)
            return tot + acc_v[...]
        tot = lax.fori_loop(0, NS, comb, jnp.zeros((16,), jnp.float32))
        acc_v[...] = tot
        pltpu.sync_copy(acc_v, s_out.at[cid])


def _reduce(h1p, degp, b1):
    mesh = plsc.VectorSubcoreMesh(
        core_axis_name="c", subcore_axis_name="s",
        num_cores=NC, num_subcores=NS)
    run = functools.partial(
        pl.kernel,
        out_type=jax.ShapeDtypeStruct((NC, 16), jnp.float32),
        mesh=mesh,
        scratch_types=[
            pltpu.VMEM((RPR, D_HID), jnp.float32),
            pltpu.VMEM((RPR, D_HID), jnp.float32),
            pltpu.VMEM((NW, RPR), jnp.float32),
            pltpu.VMEM((16,), jnp.float32),
            pltpu.VMEM((16,), jnp.float32),
            pltpu.SemaphoreType.DMA,
            pltpu.MemorySpace.VMEM_SHARED((NS, 16), jnp.float32),
        ],
        compiler_params=pltpu.CompilerParams(use_tc_tiling_on_sc=False),
    )(_reduce_body)
    return run(h1p, degp, b1)


def _head_body(sp_ref, w2_ref, b2_ref, w3_ref, b3_ref, o_ref):
    s = sp_ref[0] + sp_ref[1]
    t = jnp.dot(s[None, :], w2_ref[...], preferred_element_type=jnp.float32)
    t = t + jnp.float32(N) * b2_ref[...]
    o = jnp.dot(t, w3_ref[...], preferred_element_type=jnp.float32) + b3_ref[...]
    o_ref[...] = jnp.maximum(o, 0.0)


def _head(sp, W2, b2, W3, b3):
    return pl.pallas_call(
        _head_body,
        out_shape=jax.ShapeDtypeStruct((1, D_OUT), jnp.float32),
    )(sp, W2, b2, W3, b3)


def kernel(features, edge_index, W1, b1, W2, b2, W3, b3):
    f1 = _project(features, W1)
    h1p, degp = _edge_pass(f1, edge_index)
    sp = _reduce(h1p, degp, b1)
    return _head(sp, W2, b2, W3, b3)
